# baseline (speedup 1.0000x reference)
import functools

import jax
import jax.numpy as jnp
from jax import lax
from jax.experimental import pallas as pl
from jax.experimental.pallas import tpu as pltpu
from jax.experimental.pallas import tpu_sc as plsc

_B = 16384
_A = 128
_NC = 2
_NS = 16
_NW = _NC * _NS
_RPW = _B // _NW
_G = _RPW // 16


_CHUNK_ROWS = 128
_NCHUNK = _RPW // _CHUNK_ROWS


def _sc_rowstats(action_hbm, sum_hbm, max_hbm, buf0, buf1, sums_v, maxs_v,
                 sem0, sem1):
    wid = lax.axis_index("s") * _NC + lax.axis_index("c")
    base = wid * _RPW
    lanes = jax.lax.iota(jnp.int32, 16)
    bufs = (buf0, buf1)
    sems = (sem0, sem1)

    def start(c):
        return pltpu.async_copy(
            action_hbm.at[pl.ds((base + c * _CHUNK_ROWS) * _A,
                                _CHUNK_ROWS * _A)],
            bufs[c % 2], sems[c % 2])

    copies = [start(0)]
    for c in range(_NCHUNK):
        if c + 1 < _NCHUNK:
            copies.append(start(c + 1))
        copies[c].wait()
        buf = bufs[c % 2]

        def group(g, carry, buf=buf, c=c):
            idx = (g * 16 + lanes) * _A + lanes
            v = plsc.load_gather(buf, [idx])
            sacc = v
            macc = v
            for t in range(1, _A):
                idx = idx + jnp.where(lanes == (_A - t), jnp.int32(1 - _A),
                                      jnp.int32(1))
                v = plsc.load_gather(buf, [idx])
                sacc = sacc + v
                macc = jnp.maximum(macc, v)
            sums_v[pl.ds(c * _CHUNK_ROWS + g * 16, 16)] = sacc
            maxs_v[pl.ds(c * _CHUNK_ROWS + g * 16, 16)] = macc
            return carry

        lax.fori_loop(0, _CHUNK_ROWS // 16, group, 0)
    pltpu.sync_copy(sums_v, sum_hbm.at[pl.ds(base, _RPW)])
    pltpu.sync_copy(maxs_v, max_hbm.at[pl.ds(base, _RPW)])


_sc_call = pl.kernel(
    _sc_rowstats,
    out_type=(
        jax.ShapeDtypeStruct((_B,), jnp.float32),
        jax.ShapeDtypeStruct((_B,), jnp.float32),
    ),
    mesh=plsc.VectorSubcoreMesh(core_axis_name="c", subcore_axis_name="s"),
    compiler_params=pltpu.CompilerParams(needs_layout_passes=False),
    scratch_types=[
        pltpu.VMEM((_CHUNK_ROWS * _A,), jnp.float32),
        pltpu.VMEM((_CHUNK_ROWS * _A,), jnp.float32),
        pltpu.VMEM((_RPW,), jnp.float32),
        pltpu.VMEM((_RPW,), jnp.float32),
        pltpu.SemaphoreType.DMA,
        pltpu.SemaphoreType.DMA,
    ],
)


def _finish_kernel(label_ref, sum_ref, max_ref, out_ref):
    factor = jnp.where(label_ref[0] == 1, jnp.float32(1.25), jnp.float32(0.9))
    s = sum_ref[...] + max_ref[...] * (factor - 1.0)
    ell = jnp.log(s)
    logf = jnp.log(factor)
    a = jnp.float32(_A)
    b = jnp.float32(_B)
    out_ref[0] = (a * jnp.sum(ell * ell) - 2.0 * logf * jnp.sum(ell)
                  + b * logf * logf) / (a * b)


@jax.jit
def _run(action, label_i32):
    rowsum, rowmax = _sc_call(action.reshape(_B * _A))
    out = pl.pallas_call(
        _finish_kernel,
        in_specs=[
            pl.BlockSpec(memory_space=pltpu.SMEM),
            pl.BlockSpec((_B // _A, _A), lambda: (0, 0)),
            pl.BlockSpec((_B // _A, _A), lambda: (0, 0)),
        ],
        out_specs=pl.BlockSpec(memory_space=pltpu.SMEM),
        out_shape=jax.ShapeDtypeStruct((1,), jnp.float32),
    )(label_i32, rowsum.reshape(_B // _A, _A), rowmax.reshape(_B // _A, _A))
    return out[0]


def kernel(action, label):
    return _run(action, label.astype(jnp.int32))


# speedup vs baseline: 1.4292x; 1.4292x over previous
import functools

import jax
import jax.numpy as jnp
from jax import lax
from jax.experimental import pallas as pl
from jax.experimental.pallas import tpu as pltpu
from jax.experimental.pallas import tpu_sc as plsc

_B = 16384
_A = 128
_NC = 2
_NS = 16
_NW = _NC * _NS
_RPW = _B // _NW
_G = _RPW // 16


def _sc_rowstats(action_hbm, sum_hbm, max_hbm, buf, sscr, mscr,
                 sums_v, maxs_v):
    wid = lax.axis_index("s") * _NC + lax.axis_index("c")
    base = wid * _RPW
    pltpu.sync_copy(action_hbm.at[pl.ds(base * _A, _RPW * _A)], buf)
    lanes = jax.lax.iota(jnp.int32, 16)
    idx17 = lanes * 17

    def group(g, carry):
        rowbase = g * (16 * _A)
        for r in range(16):
            off = rowbase + r * _A
            v = [buf[pl.ds(off + k * 16, 16)] for k in range(8)]
            s01, s23 = v[0] + v[1], v[2] + v[3]
            s45, s67 = v[4] + v[5], v[6] + v[7]
            s = (s01 + s23) + (s45 + s67)
            m01, m23 = jnp.maximum(v[0], v[1]), jnp.maximum(v[2], v[3])
            m45, m67 = jnp.maximum(v[4], v[5]), jnp.maximum(v[6], v[7])
            m = jnp.maximum(jnp.maximum(m01, m23), jnp.maximum(m45, m67))
            sscr[pl.ds(r * 17, 16)] = s
            mscr[pl.ds(r * 17, 16)] = m
        sacc = plsc.load_gather(sscr, [idx17])
        macc = plsc.load_gather(mscr, [idx17])
        for t in range(1, 16):
            sacc = sacc + plsc.load_gather(sscr, [idx17 + t])
            macc = jnp.maximum(macc, plsc.load_gather(mscr, [idx17 + t]))
        sums_v[pl.ds(g * 16, 16)] = sacc
        maxs_v[pl.ds(g * 16, 16)] = macc
        return carry

    lax.fori_loop(0, _G, group, 0)
    pltpu.sync_copy(sums_v, sum_hbm.at[pl.ds(base, _RPW)])
    pltpu.sync_copy(maxs_v, max_hbm.at[pl.ds(base, _RPW)])


_sc_call = pl.kernel(
    _sc_rowstats,
    out_type=(
        jax.ShapeDtypeStruct((_B,), jnp.float32),
        jax.ShapeDtypeStruct((_B,), jnp.float32),
    ),
    mesh=plsc.VectorSubcoreMesh(core_axis_name="c", subcore_axis_name="s"),
    compiler_params=pltpu.CompilerParams(needs_layout_passes=False),
    scratch_types=[
        pltpu.VMEM((_RPW * _A,), jnp.float32),
        pltpu.VMEM((16 * 17,), jnp.float32),
        pltpu.VMEM((16 * 17,), jnp.float32),
        pltpu.VMEM((_RPW,), jnp.float32),
        pltpu.VMEM((_RPW,), jnp.float32),
    ],
)


def _finish_kernel(label_ref, sum_ref, max_ref, out_ref):
    factor = jnp.where(label_ref[0] == 1, jnp.float32(1.25), jnp.float32(0.9))
    s = sum_ref[...] + max_ref[...] * (factor - 1.0)
    ell = jnp.log(s)
    logf = jnp.log(factor)
    a = jnp.float32(_A)
    b = jnp.float32(_B)
    out_ref[0] = (a * jnp.sum(ell * ell) - 2.0 * logf * jnp.sum(ell)
                  + b * logf * logf) / (a * b)


@jax.jit
def _run(action, label_i32):
    rowsum, rowmax = _sc_call(action.reshape(_B * _A))
    out = pl.pallas_call(
        _finish_kernel,
        in_specs=[
            pl.BlockSpec(memory_space=pltpu.SMEM),
            pl.BlockSpec((_B // _A, _A), lambda: (0, 0)),
            pl.BlockSpec((_B // _A, _A), lambda: (0, 0)),
        ],
        out_specs=pl.BlockSpec(memory_space=pltpu.SMEM),
        out_shape=jax.ShapeDtypeStruct((1,), jnp.float32),
    )(label_i32, rowsum.reshape(_B // _A, _A), rowmax.reshape(_B // _A, _A))
    return out[0]


def kernel(action, label):
    return _run(action, label.astype(jnp.int32))


# speedup vs baseline: 1.5169x; 1.0614x over previous
import functools

import jax
import jax.numpy as jnp
from jax import lax
from jax.experimental import pallas as pl
from jax.experimental.pallas import tpu as pltpu
from jax.experimental.pallas import tpu_sc as plsc

_B = 16384
_A = 128
_NC = 2
_NS = 16
_NW = _NC * _NS
_RPW = _B // _NW
_G = _RPW // 16


_CH_ROWS = 128
_CHW = _CH_ROWS * _A
_GPC = _CH_ROWS // 16
_NCH = _RPW // _CH_ROWS


def _sc_rowstats(action_hbm, sum_hbm, max_hbm, buf, sscr, mscr,
                 sums_v, maxs_v, sem0, sem1):
    wid = lax.axis_index("s") * _NC + lax.axis_index("c")
    base = wid * _RPW
    lanes = jax.lax.iota(jnp.int32, 16)
    idx17 = lanes * 17

    def issue(c):
        src = action_hbm.at[pl.ds((base + c * _CH_ROWS) * _A, _CHW)]

        @pl.when(c % 2 == 0)
        def _():
            pltpu.async_copy(src, buf.at[pl.ds(0, _CHW)], sem0)

        @pl.when(c % 2 == 1)
        def _():
            pltpu.async_copy(src, buf.at[pl.ds(_CHW, _CHW)], sem1)

    def wait(c):
        src = action_hbm.at[pl.ds((base + c * _CH_ROWS) * _A, _CHW)]

        @pl.when(c % 2 == 0)
        def _():
            pltpu.make_async_copy(src, buf.at[pl.ds(0, _CHW)], sem0).wait()

        @pl.when(c % 2 == 1)
        def _():
            pltpu.make_async_copy(src, buf.at[pl.ds(_CHW, _CHW)], sem1).wait()

    issue(0)

    def group(g, carry):
        c = g // _GPC

        @pl.when(g % _GPC == 0)
        def _():
            wait(c)

            @pl.when(c + 1 < _NCH)
            def _():
                issue(c + 1)

        rowbase = (c % 2) * _CHW + (g % _GPC) * (16 * _A)
        for r in range(16):
            off = rowbase + r * _A
            v = [buf[pl.ds(off + k * 16, 16)] for k in range(8)]
            s01, s23 = v[0] + v[1], v[2] + v[3]
            s45, s67 = v[4] + v[5], v[6] + v[7]
            s = (s01 + s23) + (s45 + s67)
            m01, m23 = jnp.maximum(v[0], v[1]), jnp.maximum(v[2], v[3])
            m45, m67 = jnp.maximum(v[4], v[5]), jnp.maximum(v[6], v[7])
            m = jnp.maximum(jnp.maximum(m01, m23), jnp.maximum(m45, m67))
            sscr[pl.ds(r * 17, 16)] = s
            mscr[pl.ds(r * 17, 16)] = m
        sp = [plsc.load_gather(sscr, [idx17 + t]) for t in range(4)]
        mp = [plsc.load_gather(mscr, [idx17 + t]) for t in range(4)]
        for t in range(4, 16):
            sp[t % 4] = sp[t % 4] + plsc.load_gather(sscr, [idx17 + t])
            mp[t % 4] = jnp.maximum(mp[t % 4], plsc.load_gather(mscr, [idx17 + t]))
        sacc = (sp[0] + sp[1]) + (sp[2] + sp[3])
        macc = jnp.maximum(jnp.maximum(mp[0], mp[1]), jnp.maximum(mp[2], mp[3]))
        sums_v[pl.ds(g * 16, 16)] = sacc
        maxs_v[pl.ds(g * 16, 16)] = macc
        return carry

    lax.fori_loop(0, _G, group, 0)
    pltpu.sync_copy(sums_v, sum_hbm.at[pl.ds(base, _RPW)])
    pltpu.sync_copy(maxs_v, max_hbm.at[pl.ds(base, _RPW)])


_sc_call = pl.kernel(
    _sc_rowstats,
    out_type=(
        jax.ShapeDtypeStruct((_B,), jnp.float32),
        jax.ShapeDtypeStruct((_B,), jnp.float32),
    ),
    mesh=plsc.VectorSubcoreMesh(core_axis_name="c", subcore_axis_name="s"),
    compiler_params=pltpu.CompilerParams(needs_layout_passes=False,
                                         skip_device_barrier=True),
    scratch_types=[
        pltpu.VMEM((2 * _CHW,), jnp.float32),
        pltpu.VMEM((16 * 17,), jnp.float32),
        pltpu.VMEM((16 * 17,), jnp.float32),
        pltpu.VMEM((_RPW,), jnp.float32),
        pltpu.VMEM((_RPW,), jnp.float32),
        pltpu.SemaphoreType.DMA,
        pltpu.SemaphoreType.DMA,
    ],
)


def _finish_kernel(label_ref, sum_ref, max_ref, out_ref):
    factor = jnp.where(label_ref[0] == 1, jnp.float32(1.25), jnp.float32(0.9))
    s = sum_ref[...] + max_ref[...] * (factor - 1.0)
    ell = jnp.log(s)
    logf = jnp.log(factor)
    a = jnp.float32(_A)
    b = jnp.float32(_B)
    out_ref[0] = (a * jnp.sum(ell * ell) - 2.0 * logf * jnp.sum(ell)
                  + b * logf * logf) / (a * b)


@jax.jit
def _run(action, label_i32):
    rowsum, rowmax = _sc_call(action.reshape(_B * _A))
    out = pl.pallas_call(
        _finish_kernel,
        in_specs=[
            pl.BlockSpec(memory_space=pltpu.SMEM),
            pl.BlockSpec((_B // _A, _A), lambda: (0, 0)),
            pl.BlockSpec((_B // _A, _A), lambda: (0, 0)),
        ],
        out_specs=pl.BlockSpec(memory_space=pltpu.SMEM),
        out_shape=jax.ShapeDtypeStruct((1,), jnp.float32),
    )(label_i32, rowsum.reshape(_B // _A, _A), rowmax.reshape(_B // _A, _A))
    return out[0]


def kernel(action, label):
    return _run(action, label.astype(jnp.int32))
